# Initial kernel scaffold; baseline (speedup 1.0000x reference)
import jax
import jax.numpy as jnp
from jax.experimental import pallas as pl


def kernel(x_nodes, x_edges, edge_index, batch, params):
    raise NotImplementedError("write your pallas kernel here")


# speedup vs baseline: 2.4195x; 2.4195x over previous
import functools

import jax
import jax.numpy as jnp
from jax import lax
from jax.experimental import pallas as pl
from jax.experimental.pallas import tpu as pltpu
from jax.experimental.pallas import tpu_sc as plsc

N = 10000
E = 320000
D = 128
DE = 16
G = 64

NC = 2
NS = 16
NW = NC * NS
EPW = E // NW
C = 80
NCH = EPW // C
NPT = N // NS
ZROWS = 125
FR = 1000

BN = 2000
BE = 2560

_F32 = jnp.float32


def _full_spec(shape):
    return pl.BlockSpec(shape, lambda i: (0,) * len(shape))


def _dot(a, b):
    return jnp.dot(a, b, preferred_element_type=_F32)


def _silu(x):
    return x * jax.nn.sigmoid(x)


def _embed_body(x, w1, b1, w2, b2, wa, wb, h, p1, p2):
    t = _silu(_dot(x[...], w1[...]) + b1[...])
    hh = _dot(t, w2[...]) + b2[...]
    h[...] = hh
    p1[...] = _dot(hh, wa[...])
    p2[...] = _dot(hh, wb[...])


def _embed_call(x, w1, b1, w2, b2, wa, wb):
    return pl.pallas_call(
        _embed_body,
        grid=(N // BN,),
        in_specs=[pl.BlockSpec((BN, D), lambda i: (i, 0)),
                  _full_spec((D, D)), _full_spec((1, D)),
                  _full_spec((D, D)), _full_spec((1, D)),
                  _full_spec((D, D)), _full_spec((D, D))],
        out_specs=[pl.BlockSpec((BN, D), lambda i: (i, 0))] * 3,
        out_shape=[jax.ShapeDtypeStruct((N, D), _F32)] * 3,
    )(x, w1, b1, w2, b2, wa, wb)


def _edge_body(s1, s2, xe, w1c, b1, w2, b2, m):
    t = _silu(s1[...] + s2[...] + _dot(xe[...], w1c[...]) + b1[...])
    m[...] = _silu(_dot(t, w2[...]) + b2[...])


def _edge_call(s1, s2, xe, w1c, b1, w2, b2):
    return pl.pallas_call(
        _edge_body,
        grid=(E // BE,),
        in_specs=[pl.BlockSpec((BE, D), lambda i: (i, 0)),
                  pl.BlockSpec((BE, D), lambda i: (i, 0)),
                  pl.BlockSpec((BE, DE), lambda i: (i, 0)),
                  _full_spec((DE, D)), _full_spec((1, D)),
                  _full_spec((D, D)), _full_spec((1, D))],
        out_specs=pl.BlockSpec((BE, D), lambda i: (i, 0)),
        out_shape=jax.ShapeDtypeStruct((E, D), _F32),
    )(s1, s2, xe, w1c, b1, w2, b2)


def _node_proj_body(h, pa, pb, wn1a, wn1b, bn1, wn2, bn2, wa, wb,
                    ho, p1, p2):
    aggr = pa[...] + pb[...]
    u = _silu(_dot(h[...], wn1a[...]) + _dot(aggr, wn1b[...]) + bn1[...])
    hh = _dot(u, wn2[...]) + bn2[...]
    ho[...] = hh
    p1[...] = _dot(hh, wa[...])
    p2[...] = _dot(hh, wb[...])


def _node_proj_call(h, pa, pb, wn1a, wn1b, bn1, wn2, bn2, wa, wb):
    return pl.pallas_call(
        _node_proj_body,
        grid=(N // BN,),
        in_specs=[pl.BlockSpec((BN, D), lambda i: (i, 0))] * 3 + [
            _full_spec((D, D)), _full_spec((D, D)), _full_spec((1, D)),
            _full_spec((D, D)), _full_spec((1, D)),
            _full_spec((D, D)), _full_spec((D, D))],
        out_specs=[pl.BlockSpec((BN, D), lambda i: (i, 0))] * 3,
        out_shape=[jax.ShapeDtypeStruct((N, D), _F32)] * 3,
    )(h, pa, pb, wn1a, wn1b, bn1, wn2, bn2, wa, wb)


def _node_last_body(h, pa, pb, wn1a, wn1b, bn1, wn2, bn2, ho):
    aggr = pa[...] + pb[...]
    u = _silu(_dot(h[...], wn1a[...]) + _dot(aggr, wn1b[...]) + bn1[...])
    ho[...] = _dot(u, wn2[...]) + bn2[...]


def _node_last_call(h, pa, pb, wn1a, wn1b, bn1, wn2, bn2):
    return pl.pallas_call(
        _node_last_body,
        grid=(N // BN,),
        in_specs=[pl.BlockSpec((BN, D), lambda i: (i, 0))] * 3 + [
            _full_spec((D, D)), _full_spec((D, D)), _full_spec((1, D)),
            _full_spec((D, D)), _full_spec((1, D))],
        out_specs=pl.BlockSpec((BN, D), lambda i: (i, 0)),
        out_shape=jax.ShapeDtypeStruct((N, D), _F32),
    )(h, pa, pb, wn1a, wn1b, bn1, wn2, bn2)


def _prepool_body(h, w1, b1, w2, b2, bat, o):
    t = _silu(_dot(h[...], w1[...]) + b1[...])
    hp = _dot(t, w2[...]) + b2[...]
    ids = bat[0]
    iota = lax.broadcasted_iota(jnp.int32, (G, BN), 0).astype(_F32)
    onehot = (iota == ids).astype(_F32)
    part = _dot(onehot, hp)

    @pl.when(pl.program_id(0) == 0)
    def _init():
        o[...] = jnp.zeros_like(o)

    o[...] += part


def _prepool_call(h, w1, b1, w2, b2, batf):
    return pl.pallas_call(
        _prepool_body,
        grid=(N // BN,),
        in_specs=[pl.BlockSpec((BN, D), lambda i: (i, 0)),
                  _full_spec((D, D)), _full_spec((1, D)),
                  _full_spec((D, D)), _full_spec((1, D)),
                  pl.BlockSpec((1, 1, BN), lambda i: (i, 0, 0))],
        out_specs=pl.BlockSpec((G, D), lambda i: (0, 0)),
        out_shape=jax.ShapeDtypeStruct((G, D), _F32),
    )(h, w1, b1, w2, b2, batf)


def _readout_body(g, w1, b1, w2, b2, o):
    u = _silu(_dot(g[...], w1[...]) + b1[...])
    o[...] = _dot(u, w2[...]) + b2[...]


def _readout_call(g, w1, b1, w2, b2):
    return pl.pallas_call(
        _readout_body,
        grid=(1,),
        in_specs=[_full_spec((G, D)),
                  _full_spec((D, D)), _full_spec((1, D)),
                  _full_spec((D, 1)), _full_spec((1, 1))],
        out_specs=_full_spec((G, 1)),
        out_shape=jax.ShapeDtypeStruct((G, 1), _F32),
    )(g, w1, b1, w2, b2)


@functools.cache
def _mesh():
    return plsc.VectorSubcoreMesh(core_axis_name="c", subcore_axis_name="s",
                                  num_cores=NC, num_subcores=NS)


def _sc_gather(p1, p2, snd, rcv):
    @functools.partial(
        pl.kernel,
        out_type=[jax.ShapeDtypeStruct((E, D), _F32),
                  jax.ShapeDtypeStruct((E, D), _F32)],
        mesh=_mesh(),
        scratch_types=[pltpu.VMEM((C,), jnp.int32),
                       pltpu.VMEM((C,), jnp.int32),
                       pltpu.VMEM((C, D), _F32),
                       pltpu.VMEM((C, D), _F32),
                       pltpu.SemaphoreType.DMA,
                       pltpu.SemaphoreType.DMA],
    )
    def k(p1_h, p2_h, snd_h, rcv_h, s1_h, s2_h,
          idx1, idx2, rows1, rows2, sem1, sem2):
        wid = lax.axis_index("s") * NC + lax.axis_index("c")
        base = wid * EPW

        def body(i, carry):
            off = base + i * C
            pltpu.sync_copy(snd_h.at[pl.ds(off, C)], idx1)
            pltpu.sync_copy(rcv_h.at[pl.ds(off, C)], idx2)
            cp1 = pltpu.async_copy(p1_h.at[idx1], rows1, sem1)
            cp2 = pltpu.async_copy(p2_h.at[idx2], rows2, sem2)
            cp1.wait()
            cp2.wait()
            pltpu.sync_copy(rows1, s1_h.at[pl.ds(off, C)])
            pltpu.sync_copy(rows2, s2_h.at[pl.ds(off, C)])
            return carry

        lax.fori_loop(0, NCH, body, 0)

    return k(p1, p2, snd, rcv)


def _sc_scatter(m, rcv):
    @functools.partial(
        pl.kernel,
        out_type=jax.ShapeDtypeStruct((NC * N, D), _F32),
        mesh=_mesh(),
        scratch_types=[pltpu.VMEM((C,), jnp.int32),
                       pltpu.VMEM((C, D), _F32),
                       pltpu.VMEM((ZROWS, D), _F32),
                       pltpu.VMEM_SHARED((N, D), _F32)],
    )
    def k(m_h, rcv_h, out_h, idx, rows, zbuf, aggr):
        cid = lax.axis_index("c")
        sid = lax.axis_index("s")
        base = (sid * NC + cid) * EPW

        def zrow(r, carry):
            def zcol(c2, carry2):
                zbuf[r, pl.ds(c2 * 16, 16)] = jnp.zeros((16,), _F32)
                return carry2
            return lax.fori_loop(0, D // 16, zcol, carry)

        lax.fori_loop(0, ZROWS, zrow, 0)

        def zcopy(j, carry):
            pltpu.sync_copy(zbuf, aggr.at[pl.ds(sid * NPT + j * ZROWS, ZROWS)])
            return carry

        lax.fori_loop(0, NPT // ZROWS, zcopy, 0)
        plsc.subcore_barrier()

        def body(i, carry):
            off = base + i * C
            pltpu.sync_copy(rcv_h.at[pl.ds(off, C)], idx)
            pltpu.sync_copy(m_h.at[pl.ds(off, C)], rows)
            pltpu.sync_copy(rows, aggr.at[idx], add=True)
            return carry

        lax.fori_loop(0, NCH, body, 0)
        plsc.subcore_barrier()

        @pl.when(sid < N // FR)
        def _flush():
            pltpu.sync_copy(aggr.at[pl.ds(sid * FR, FR)],
                            out_h.at[pl.ds(cid * N + sid * FR, FR)])

    return k(m, rcv)


def kernel(x_nodes, x_edges, edge_index, batch, params):
    sender = edge_index[0]
    receiver = edge_index[1]
    layers = params['layers']

    e0_w, e0_b = params['embed'][0]
    e1_w, e1_b = params['embed'][1]
    w1_first = layers[0]['edge'][0][0]
    h, p1, p2 = _embed_call(
        x_nodes, e0_w, e0_b.reshape(1, D), e1_w, e1_b.reshape(1, D),
        w1_first[:D], w1_first[D:2 * D])

    for li, lp in enumerate(layers):
        we1, be1 = lp['edge'][0]
        we2, be2 = lp['edge'][1]
        s1, s2 = _sc_gather(p1, p2, sender, receiver)
        m = _edge_call(s1, s2, x_edges, we1[2 * D:], be1.reshape(1, D),
                       we2, be2.reshape(1, D))
        parts = _sc_scatter(m, receiver).reshape(2, N, D)
        wn1, bn1 = lp['node'][0]
        wn2, bn2 = lp['node'][1]
        if li + 1 < len(layers):
            w1_next = layers[li + 1]['edge'][0][0]
            h, p1, p2 = _node_proj_call(
                h, parts[0], parts[1], wn1[:D], wn1[D:], bn1.reshape(1, D),
                wn2, bn2.reshape(1, D), w1_next[:D], w1_next[D:2 * D])
        else:
            h = _node_last_call(
                h, parts[0], parts[1], wn1[:D], wn1[D:], bn1.reshape(1, D),
                wn2, bn2.reshape(1, D))

    pr0_w, pr0_b = params['pre'][0]
    pr1_w, pr1_b = params['pre'][1]
    batf = batch.astype(_F32).reshape(N // BN, 1, BN)
    g = _prepool_call(h, pr0_w, pr0_b.reshape(1, D),
                      pr1_w, pr1_b.reshape(1, D), batf)

    r0_w, r0_b = params['readout'][0]
    r1_w, r1_b = params['readout'][1]
    out = _readout_call(g, r0_w, r0_b.reshape(1, D),
                        r1_w, r1_b.reshape(1, 1))
    return out.reshape(G)


# speedup vs baseline: 3.3630x; 1.3899x over previous
import functools

import jax
import jax.numpy as jnp
from jax import lax
from jax.experimental import pallas as pl
from jax.experimental.pallas import tpu as pltpu
from jax.experimental.pallas import tpu_sc as plsc

N = 10000
E = 320000
D = 128
DE = 16
G = 64

NC = 2
NS = 16
NW = NC * NS
EPW = E // NW
C = 80
NCH = EPW // C
NPT = N // NS
ZROWS = 125
ZROWS_S = 25
FR = 1000
NBLG = 3
NBLS = 3
EPS = E // NS
NCHS = EPS // C
DH = D // 2

BN = 2000
BE = 2560

_F32 = jnp.float32


def _full_spec(shape):
    return pl.BlockSpec(shape, lambda i: (0,) * len(shape))


def _dot(a, b):
    return jnp.dot(a, b, preferred_element_type=_F32)


def _silu(x):
    return x * jax.nn.sigmoid(x)


def _embed_body(x, w1, b1, w2, b2, wa, wb, h, p1, p2):
    t = _silu(_dot(x[...], w1[...]) + b1[...])
    hh = _dot(t, w2[...]) + b2[...]
    h[...] = hh
    p1[...] = _dot(hh, wa[...])
    p2[...] = _dot(hh, wb[...])


def _embed_call(x, w1, b1, w2, b2, wa, wb):
    return pl.pallas_call(
        _embed_body,
        grid=(N // BN,),
        in_specs=[pl.BlockSpec((BN, D), lambda i: (i, 0)),
                  _full_spec((D, D)), _full_spec((1, D)),
                  _full_spec((D, D)), _full_spec((1, D)),
                  _full_spec((D, D)), _full_spec((D, D))],
        out_specs=[pl.BlockSpec((BN, D), lambda i: (i, 0))] * 3,
        out_shape=[jax.ShapeDtypeStruct((N, D), _F32)] * 3,
    )(x, w1, b1, w2, b2, wa, wb)


def _edge_body(s1, s2, xe, w1c, b1, w2, b2, m):
    t = _silu(s1[...] + s2[...] + _dot(xe[...], w1c[...]) + b1[...])
    m[...] = _silu(_dot(t, w2[...]) + b2[...])


def _edge_call(s1, s2, xe, w1c, b1, w2, b2):
    return pl.pallas_call(
        _edge_body,
        grid=(E // BE,),
        in_specs=[pl.BlockSpec((BE, D), lambda i: (i, 0)),
                  pl.BlockSpec((BE, D), lambda i: (i, 0)),
                  pl.BlockSpec((BE, DE), lambda i: (i, 0)),
                  _full_spec((DE, D)), _full_spec((1, D)),
                  _full_spec((D, D)), _full_spec((1, D))],
        out_specs=pl.BlockSpec((BE, D), lambda i: (i, 0)),
        out_shape=jax.ShapeDtypeStruct((E, D), _F32),
    )(s1, s2, xe, w1c, b1, w2, b2)


def _node_proj_body(h, pa, pb, wn1a, wn1b, bn1, wn2, bn2, wa, wb,
                    ho, p1, p2):
    u = _silu(_dot(h[...], wn1a[...]) + _dot(pa[...] + pb[...], wn1b[...])
              + bn1[...])
    hh = _dot(u, wn2[...]) + bn2[...]
    ho[...] = hh
    p1[...] = _dot(hh, wa[...])
    p2[...] = _dot(hh, wb[...])


def _node_proj_call(h, pa, pb, wn1a, wn1b, bn1, wn2, bn2, wa, wb):
    return pl.pallas_call(
        _node_proj_body,
        grid=(N // BN,),
        in_specs=[pl.BlockSpec((BN, D), lambda i: (i, 0))] * 3 + [
            _full_spec((D, D)), _full_spec((D, D)), _full_spec((1, D)),
            _full_spec((D, D)), _full_spec((1, D)),
            _full_spec((D, D)), _full_spec((D, D))],
        out_specs=[pl.BlockSpec((BN, D), lambda i: (i, 0))] * 3,
        out_shape=[jax.ShapeDtypeStruct((N, D), _F32)] * 3,
    )(h, pa, pb, wn1a, wn1b, bn1, wn2, bn2, wa, wb)


def _node_last_body(h, pa, pb, wn1a, wn1b, bn1, wn2, bn2, ho):
    u = _silu(_dot(h[...], wn1a[...]) + _dot(pa[...] + pb[...], wn1b[...])
              + bn1[...])
    ho[...] = _dot(u, wn2[...]) + bn2[...]


def _node_last_call(h, pa, pb, wn1a, wn1b, bn1, wn2, bn2):
    return pl.pallas_call(
        _node_last_body,
        grid=(N // BN,),
        in_specs=[pl.BlockSpec((BN, D), lambda i: (i, 0))] * 3 + [
            _full_spec((D, D)), _full_spec((D, D)), _full_spec((1, D)),
            _full_spec((D, D)), _full_spec((1, D))],
        out_specs=pl.BlockSpec((BN, D), lambda i: (i, 0)),
        out_shape=jax.ShapeDtypeStruct((N, D), _F32),
    )(h, pa, pb, wn1a, wn1b, bn1, wn2, bn2)


def _prepool_body(h, w1, b1, w2, b2, bat, o):
    t = _silu(_dot(h[...], w1[...]) + b1[...])
    hp = _dot(t, w2[...]) + b2[...]
    ids = bat[0]
    iota = lax.broadcasted_iota(jnp.int32, (G, BN), 0).astype(_F32)
    onehot = (iota == ids).astype(_F32)
    part = _dot(onehot, hp)

    @pl.when(pl.program_id(0) == 0)
    def _init():
        o[...] = jnp.zeros_like(o)

    o[...] += part


def _prepool_call(h, w1, b1, w2, b2, batf):
    return pl.pallas_call(
        _prepool_body,
        grid=(N // BN,),
        in_specs=[pl.BlockSpec((BN, D), lambda i: (i, 0)),
                  _full_spec((D, D)), _full_spec((1, D)),
                  _full_spec((D, D)), _full_spec((1, D)),
                  pl.BlockSpec((1, 1, BN), lambda i: (i, 0, 0))],
        out_specs=pl.BlockSpec((G, D), lambda i: (0, 0)),
        out_shape=jax.ShapeDtypeStruct((G, D), _F32),
    )(h, w1, b1, w2, b2, batf)


def _readout_body(g, w1, b1, w2, b2, o):
    u = _silu(_dot(g[...], w1[...]) + b1[...])
    o[...] = _dot(u, w2[...]) + b2[...]


def _readout_call(g, w1, b1, w2, b2):
    return pl.pallas_call(
        _readout_body,
        grid=(1,),
        in_specs=[_full_spec((G, D)),
                  _full_spec((D, D)), _full_spec((1, D)),
                  _full_spec((D, 1)), _full_spec((1, 1))],
        out_specs=_full_spec((G, 1)),
        out_shape=jax.ShapeDtypeStruct((G, 1), _F32),
    )(g, w1, b1, w2, b2)


@functools.cache
def _mesh():
    return plsc.VectorSubcoreMesh(core_axis_name="c", subcore_axis_name="s",
                                  num_cores=NC, num_subcores=NS)


def _sc_gather(p1, p2, snd, rcv):
    sds = jax.ShapeDtypeStruct((E, D), _F32)
    scr = []
    for _ in range(NBLG):
        scr += [pltpu.VMEM((C,), jnp.int32), pltpu.VMEM((C,), jnp.int32),
                pltpu.VMEM((C, D), _F32), pltpu.VMEM((C, D), _F32),
                pltpu.SemaphoreType.DMA, pltpu.SemaphoreType.DMA,
                pltpu.SemaphoreType.DMA]

    @functools.partial(pl.kernel, out_type=[sds, sds], mesh=_mesh(),
                       scratch_types=scr)
    def k(p1_h, p2_h, snd_h, rcv_h, s1_h, s2_h, *scratch):
        lanes = [scratch[i * 7:(i + 1) * 7] for i in range(NBLG)]
        wid = lax.axis_index("s") * NC + lax.axis_index("c")
        base = wid * EPW

        def chunk_group(first, nbl):
            ips = []
            for b in range(nbl):
                idx1, idx2, r1, r2, isem, gsem, wsem = lanes[b]
                sl = pl.ds(base + (first + b) * C, C)
                ips.append(
                    (pltpu.async_copy(snd_h.at[sl], idx1, isem),
                     pltpu.async_copy(rcv_h.at[sl], idx2, isem)))
            gps = []
            for b in range(nbl):
                idx1, idx2, r1, r2, isem, gsem, wsem = lanes[b]
                i1, i2 = ips[b]
                i1.wait()
                i2.wait()
                gps.append(
                    (pltpu.async_copy(p1_h.at[idx1], r1, gsem),
                     pltpu.async_copy(p2_h.at[idx2], r2, gsem)))
            wps = []
            for b in range(nbl):
                idx1, idx2, r1, r2, isem, gsem, wsem = lanes[b]
                g1, g2 = gps[b]
                g1.wait()
                g2.wait()
                sl = pl.ds(base + (first + b) * C, C)
                wps.append(
                    (pltpu.async_copy(r1, s1_h.at[sl], wsem),
                     pltpu.async_copy(r2, s2_h.at[sl], wsem)))
            for w1, w2 in wps:
                w1.wait()
                w2.wait()

        def body(g, carry):
            chunk_group(g * NBLG, NBLG)
            return carry

        lax.fori_loop(0, NCH // NBLG, body, 0)
        if NCH % NBLG:
            chunk_group((NCH // NBLG) * NBLG, NCH % NBLG)

    return k(p1, p2, snd, rcv)


def _sc_scatter(m, rcv):
    scr = []
    for _ in range(NBLS):
        scr += [pltpu.VMEM((C,), jnp.int32), pltpu.VMEM((C, D), _F32),
                pltpu.SemaphoreType.DMA, pltpu.SemaphoreType.DMA,
                pltpu.SemaphoreType.DMA]
    scr += [pltpu.VMEM((ZROWS_S, D), _F32),
            pltpu.VMEM_SHARED((N, D), _F32)]

    @functools.partial(pl.kernel,
                       out_type=jax.ShapeDtypeStruct((NC * N, D), _F32),
                       mesh=_mesh(), scratch_types=scr)
    def k(m_h, rcv_h, out_h, *scratch):
        lanes = [scratch[i * 5:(i + 1) * 5] for i in range(NBLS)]
        zbuf, aggr = scratch[5 * NBLS], scratch[5 * NBLS + 1]
        cid = lax.axis_index("c")
        sid = lax.axis_index("s")
        base = (sid * NC + cid) * EPW

        def zrow(r, carry):
            def zcol(c2, carry2):
                zbuf[r, pl.ds(c2 * 16, 16)] = jnp.zeros((16,), _F32)
                return carry2
            return lax.fori_loop(0, D // 16, zcol, carry)

        lax.fori_loop(0, ZROWS_S, zrow, 0)

        def zcopy(j, carry):
            pltpu.sync_copy(zbuf,
                            aggr.at[pl.ds(sid * NPT + j * ZROWS_S, ZROWS_S)])
            return carry

        lax.fori_loop(0, NPT // ZROWS_S, zcopy, 0)
        plsc.subcore_barrier()

        def chunk_group(first, nbl):
            ips = []
            for b in range(nbl):
                idx, rows, isem, rsem, ssem = lanes[b]
                sl = pl.ds(base + (first + b) * C, C)
                ips.append(
                    (pltpu.async_copy(rcv_h.at[sl], idx, isem),
                     pltpu.async_copy(m_h.at[sl], rows, rsem)))
            sps = []
            for b in range(nbl):
                idx, rows, isem, rsem, ssem = lanes[b]
                i1, r1 = ips[b]
                i1.wait()
                r1.wait()
                sps.append(pltpu.async_copy(rows, aggr.at[idx], ssem,
                                            add=True))
            for sp in sps:
                sp.wait()

        def body(g, carry):
            chunk_group(g * NBLS, NBLS)
            return carry

        lax.fori_loop(0, NCH // NBLS, body, 0)
        if NCH % NBLS:
            chunk_group((NCH // NBLS) * NBLS, NCH % NBLS)
        plsc.subcore_barrier()

        @pl.when(sid < N // FR)
        def _flush():
            pltpu.sync_copy(aggr.at[pl.ds(sid * FR, FR)],
                            out_h.at[pl.ds(cid * N + sid * FR, FR)])

    return k(m, rcv)


def kernel(x_nodes, x_edges, edge_index, batch, params):
    sender = edge_index[0]
    receiver = edge_index[1]
    layers = params['layers']

    e0_w, e0_b = params['embed'][0]
    e1_w, e1_b = params['embed'][1]
    w1_first = layers[0]['edge'][0][0]
    h, p1, p2 = _embed_call(
        x_nodes, e0_w, e0_b.reshape(1, D), e1_w, e1_b.reshape(1, D),
        w1_first[:D], w1_first[D:2 * D])

    for li, lp in enumerate(layers):
        we1, be1 = lp['edge'][0]
        we2, be2 = lp['edge'][1]
        s1, s2 = _sc_gather(p1, p2, sender, receiver)
        m = _edge_call(s1, s2, x_edges, we1[2 * D:], be1.reshape(1, D),
                       we2, be2.reshape(1, D))
        parts = _sc_scatter(m, receiver).reshape(2, N, D)
        wn1, bn1 = lp['node'][0]
        wn2, bn2 = lp['node'][1]
        if li + 1 < len(layers):
            w1_next = layers[li + 1]['edge'][0][0]
            h, p1, p2 = _node_proj_call(
                h, parts[0], parts[1], wn1[:D], wn1[D:], bn1.reshape(1, D),
                wn2, bn2.reshape(1, D), w1_next[:D], w1_next[D:2 * D])
        else:
            h = _node_last_call(
                h, parts[0], parts[1], wn1[:D], wn1[D:], bn1.reshape(1, D),
                wn2, bn2.reshape(1, D))

    pr0_w, pr0_b = params['pre'][0]
    pr1_w, pr1_b = params['pre'][1]
    batf = batch.astype(_F32).reshape(N // BN, 1, BN)
    g = _prepool_call(h, pr0_w, pr0_b.reshape(1, D),
                      pr1_w, pr1_b.reshape(1, D), batf)

    r0_w, r0_b = params['readout'][0]
    r1_w, r1_b = params['readout'][1]
    out = _readout_call(g, r0_w, r0_b.reshape(1, D),
                        r1_w, r1_b.reshape(1, 1))
    return out.reshape(G)


# speedup vs baseline: 3.5257x; 1.0484x over previous
import functools

import jax
import jax.numpy as jnp
from jax import lax
from jax.experimental import pallas as pl
from jax.experimental.pallas import tpu as pltpu
from jax.experimental.pallas import tpu_sc as plsc

N = 10000
E = 320000
D = 128
DE = 16
G = 64

NC = 2
NS = 16
NW = NC * NS
EPW = E // NW
C = 80
NCH = EPW // C
NPT = N // NS
ZROWS = 125
ZROWS_S = 25
FR = 1000
NBLG = 3
NBLS = 3
EPS = E // NS
NCHS = EPS // C
DH = D // 2

BN = 2000
BE = 2560

_F32 = jnp.float32


def _full_spec(shape):
    return pl.BlockSpec(shape, lambda i: (0,) * len(shape))


def _dot(a, b):
    return jnp.dot(a, b, preferred_element_type=_F32)


def _silu(x):
    return x * jax.nn.sigmoid(x)


def _embed_body(x, w1, b1, w2, b2, wa, wb, h, p1, p2):
    t = _silu(_dot(x[...], w1[...]) + b1[...])
    hh = _dot(t, w2[...]) + b2[...]
    h[...] = hh
    p1[...] = _dot(hh, wa[...])
    p2[...] = _dot(hh, wb[...])


def _embed_call(x, w1, b1, w2, b2, wa, wb):
    return pl.pallas_call(
        _embed_body,
        grid=(N // BN,),
        in_specs=[pl.BlockSpec((BN, D), lambda i: (i, 0)),
                  _full_spec((D, D)), _full_spec((1, D)),
                  _full_spec((D, D)), _full_spec((1, D)),
                  _full_spec((D, D)), _full_spec((D, D))],
        out_specs=[pl.BlockSpec((BN, D), lambda i: (i, 0))] * 3,
        out_shape=[jax.ShapeDtypeStruct((N, D), _F32)] * 3,
    )(x, w1, b1, w2, b2, wa, wb)


def _edge_body(s, xe, w1c, b1, w2, b2, m):
    t = _silu(s[...] + _dot(xe[...], w1c[...]) + b1[...])
    m[...] = _silu(_dot(t, w2[...]) + b2[...])


def _edge_call(s, xe, w1c, b1, w2, b2):
    return pl.pallas_call(
        _edge_body,
        grid=(E // BE,),
        in_specs=[pl.BlockSpec((BE, D), lambda i: (i, 0)),
                  pl.BlockSpec((BE, DE), lambda i: (i, 0)),
                  _full_spec((DE, D)), _full_spec((1, D)),
                  _full_spec((D, D)), _full_spec((1, D))],
        out_specs=pl.BlockSpec((BE, D), lambda i: (i, 0)),
        out_shape=jax.ShapeDtypeStruct((E, D), _F32),
    )(s, xe, w1c, b1, w2, b2)


def _node_proj_body(h, pa, pb, wn1a, wn1b, bn1, wn2, bn2, wa, wb,
                    ho, p1, p2):
    u = _silu(_dot(h[...], wn1a[...]) + _dot(pa[...] + pb[...], wn1b[...])
              + bn1[...])
    hh = _dot(u, wn2[...]) + bn2[...]
    ho[...] = hh
    p1[...] = _dot(hh, wa[...])
    p2[...] = _dot(hh, wb[...])


def _node_proj_call(h, pa, pb, wn1a, wn1b, bn1, wn2, bn2, wa, wb):
    return pl.pallas_call(
        _node_proj_body,
        grid=(N // BN,),
        in_specs=[pl.BlockSpec((BN, D), lambda i: (i, 0))] * 3 + [
            _full_spec((D, D)), _full_spec((D, D)), _full_spec((1, D)),
            _full_spec((D, D)), _full_spec((1, D)),
            _full_spec((D, D)), _full_spec((D, D))],
        out_specs=[pl.BlockSpec((BN, D), lambda i: (i, 0))] * 3,
        out_shape=[jax.ShapeDtypeStruct((N, D), _F32)] * 3,
    )(h, pa, pb, wn1a, wn1b, bn1, wn2, bn2, wa, wb)


def _node_last_body(h, pa, pb, wn1a, wn1b, bn1, wn2, bn2, ho):
    u = _silu(_dot(h[...], wn1a[...]) + _dot(pa[...] + pb[...], wn1b[...])
              + bn1[...])
    ho[...] = _dot(u, wn2[...]) + bn2[...]


def _node_last_call(h, pa, pb, wn1a, wn1b, bn1, wn2, bn2):
    return pl.pallas_call(
        _node_last_body,
        grid=(N // BN,),
        in_specs=[pl.BlockSpec((BN, D), lambda i: (i, 0))] * 3 + [
            _full_spec((D, D)), _full_spec((D, D)), _full_spec((1, D)),
            _full_spec((D, D)), _full_spec((1, D))],
        out_specs=pl.BlockSpec((BN, D), lambda i: (i, 0)),
        out_shape=jax.ShapeDtypeStruct((N, D), _F32),
    )(h, pa, pb, wn1a, wn1b, bn1, wn2, bn2)


def _prepool_body(h, w1, b1, w2, b2, bat, o):
    t = _silu(_dot(h[...], w1[...]) + b1[...])
    hp = _dot(t, w2[...]) + b2[...]
    ids = bat[0]
    iota = lax.broadcasted_iota(jnp.int32, (G, BN), 0).astype(_F32)
    onehot = (iota == ids).astype(_F32)
    part = _dot(onehot, hp)

    @pl.when(pl.program_id(0) == 0)
    def _init():
        o[...] = jnp.zeros_like(o)

    o[...] += part


def _prepool_call(h, w1, b1, w2, b2, batf):
    return pl.pallas_call(
        _prepool_body,
        grid=(N // BN,),
        in_specs=[pl.BlockSpec((BN, D), lambda i: (i, 0)),
                  _full_spec((D, D)), _full_spec((1, D)),
                  _full_spec((D, D)), _full_spec((1, D)),
                  pl.BlockSpec((1, 1, BN), lambda i: (i, 0, 0))],
        out_specs=pl.BlockSpec((G, D), lambda i: (0, 0)),
        out_shape=jax.ShapeDtypeStruct((G, D), _F32),
    )(h, w1, b1, w2, b2, batf)


def _readout_body(g, w1, b1, w2, b2, o):
    u = _silu(_dot(g[...], w1[...]) + b1[...])
    o[...] = _dot(u, w2[...]) + b2[...]


def _readout_call(g, w1, b1, w2, b2):
    return pl.pallas_call(
        _readout_body,
        grid=(1,),
        in_specs=[_full_spec((G, D)),
                  _full_spec((D, D)), _full_spec((1, D)),
                  _full_spec((D, 1)), _full_spec((1, 1))],
        out_specs=_full_spec((G, 1)),
        out_shape=jax.ShapeDtypeStruct((G, 1), _F32),
    )(g, w1, b1, w2, b2)


@functools.cache
def _mesh():
    return plsc.VectorSubcoreMesh(core_axis_name="c", subcore_axis_name="s",
                                  num_cores=NC, num_subcores=NS)


def _sc_gather(p1, p2, snd, rcv):
    sds = jax.ShapeDtypeStruct((E, D), _F32)
    scr = []
    for _ in range(NBLG):
        scr += [pltpu.VMEM((C,), jnp.int32), pltpu.VMEM((C,), jnp.int32),
                pltpu.VMEM((C, D), _F32), pltpu.VMEM((C, D), _F32),
                pltpu.SemaphoreType.DMA, pltpu.SemaphoreType.DMA,
                pltpu.SemaphoreType.DMA]

    @functools.partial(pl.kernel, out_type=sds, mesh=_mesh(),
                       scratch_types=scr)
    def k(p1_h, p2_h, snd_h, rcv_h, s_h, *scratch):
        lanes = [scratch[i * 7:(i + 1) * 7] for i in range(NBLG)]
        wid = lax.axis_index("s") * NC + lax.axis_index("c")
        base = wid * EPW

        def chunk_group(first, nbl):
            ips = []
            for b in range(nbl):
                idx1, idx2, r1, r2, isem, gsem, wsem = lanes[b]
                sl = pl.ds(base + (first + b) * C, C)
                ips.append(
                    (pltpu.async_copy(snd_h.at[sl], idx1, isem),
                     pltpu.async_copy(rcv_h.at[sl], idx2, isem)))
            gps = []
            for b in range(nbl):
                idx1, idx2, r1, r2, isem, gsem, wsem = lanes[b]
                i1, i2 = ips[b]
                i1.wait()
                i2.wait()
                gps.append(
                    (pltpu.async_copy(p1_h.at[idx1], r1, gsem),
                     pltpu.async_copy(p2_h.at[idx2], r2, gsem)))
            wps = []
            for b in range(nbl):
                idx1, idx2, r1, r2, isem, gsem, wsem = lanes[b]
                g1, g2 = gps[b]
                g1.wait()
                g2.wait()

                def add_row(r, carry):
                    for c2 in range(D // 16):
                        cs = pl.ds(c2 * 16, 16)
                        r1[r, cs] = r1[r, cs] + r2[r, cs]
                    return carry

                lax.fori_loop(0, C, add_row, 0)
                sl = pl.ds(base + (first + b) * C, C)
                wps.append(pltpu.async_copy(r1, s_h.at[sl], wsem))
            for w1 in wps:
                w1.wait()

        def body(g, carry):
            chunk_group(g * NBLG, NBLG)
            return carry

        lax.fori_loop(0, NCH // NBLG, body, 0)
        if NCH % NBLG:
            chunk_group((NCH // NBLG) * NBLG, NCH % NBLG)

    return k(p1, p2, snd, rcv)


def _sc_scatter(m, rcv):
    scr = []
    for _ in range(NBLS):
        scr += [pltpu.VMEM((C,), jnp.int32), pltpu.VMEM((C, D), _F32),
                pltpu.SemaphoreType.DMA, pltpu.SemaphoreType.DMA,
                pltpu.SemaphoreType.DMA]
    scr += [pltpu.VMEM((ZROWS_S, D), _F32),
            pltpu.VMEM_SHARED((N, D), _F32)]

    @functools.partial(pl.kernel,
                       out_type=jax.ShapeDtypeStruct((NC * N, D), _F32),
                       mesh=_mesh(), scratch_types=scr)
    def k(m_h, rcv_h, out_h, *scratch):
        lanes = [scratch[i * 5:(i + 1) * 5] for i in range(NBLS)]
        zbuf, aggr = scratch[5 * NBLS], scratch[5 * NBLS + 1]
        cid = lax.axis_index("c")
        sid = lax.axis_index("s")
        base = (sid * NC + cid) * EPW

        def zrow(r, carry):
            def zcol(c2, carry2):
                zbuf[r, pl.ds(c2 * 16, 16)] = jnp.zeros((16,), _F32)
                return carry2
            return lax.fori_loop(0, D // 16, zcol, carry)

        lax.fori_loop(0, ZROWS_S, zrow, 0)

        def zcopy(j, carry):
            pltpu.sync_copy(zbuf,
                            aggr.at[pl.ds(sid * NPT + j * ZROWS_S, ZROWS_S)])
            return carry

        lax.fori_loop(0, NPT // ZROWS_S, zcopy, 0)
        plsc.subcore_barrier()

        def chunk_group(first, nbl):
            ips = []
            for b in range(nbl):
                idx, rows, isem, rsem, ssem = lanes[b]
                sl = pl.ds(base + (first + b) * C, C)
                ips.append(
                    (pltpu.async_copy(rcv_h.at[sl], idx, isem),
                     pltpu.async_copy(m_h.at[sl], rows, rsem)))
            sps = []
            for b in range(nbl):
                idx, rows, isem, rsem, ssem = lanes[b]
                i1, r1 = ips[b]
                i1.wait()
                r1.wait()
                sps.append(pltpu.async_copy(rows, aggr.at[idx], ssem,
                                            add=True))
            for sp in sps:
                sp.wait()

        def body(g, carry):
            chunk_group(g * NBLS, NBLS)
            return carry

        lax.fori_loop(0, NCH // NBLS, body, 0)
        if NCH % NBLS:
            chunk_group((NCH // NBLS) * NBLS, NCH % NBLS)
        plsc.subcore_barrier()

        @pl.when(sid < N // FR)
        def _flush():
            pltpu.sync_copy(aggr.at[pl.ds(sid * FR, FR)],
                            out_h.at[pl.ds(cid * N + sid * FR, FR)])

    return k(m, rcv)


def kernel(x_nodes, x_edges, edge_index, batch, params):
    sender = edge_index[0]
    receiver = edge_index[1]
    layers = params['layers']

    e0_w, e0_b = params['embed'][0]
    e1_w, e1_b = params['embed'][1]
    w1_first = layers[0]['edge'][0][0]
    h, p1, p2 = _embed_call(
        x_nodes, e0_w, e0_b.reshape(1, D), e1_w, e1_b.reshape(1, D),
        w1_first[:D], w1_first[D:2 * D])

    for li, lp in enumerate(layers):
        we1, be1 = lp['edge'][0]
        we2, be2 = lp['edge'][1]
        s = _sc_gather(p1, p2, sender, receiver)
        m = _edge_call(s, x_edges, we1[2 * D:], be1.reshape(1, D),
                       we2, be2.reshape(1, D))
        parts = _sc_scatter(m, receiver).reshape(2, N, D)
        wn1, bn1 = lp['node'][0]
        wn2, bn2 = lp['node'][1]
        if li + 1 < len(layers):
            w1_next = layers[li + 1]['edge'][0][0]
            h, p1, p2 = _node_proj_call(
                h, parts[0], parts[1], wn1[:D], wn1[D:], bn1.reshape(1, D),
                wn2, bn2.reshape(1, D), w1_next[:D], w1_next[D:2 * D])
        else:
            h = _node_last_call(
                h, parts[0], parts[1], wn1[:D], wn1[D:], bn1.reshape(1, D),
                wn2, bn2.reshape(1, D))

    pr0_w, pr0_b = params['pre'][0]
    pr1_w, pr1_b = params['pre'][1]
    batf = batch.astype(_F32).reshape(N // BN, 1, BN)
    g = _prepool_call(h, pr0_w, pr0_b.reshape(1, D),
                      pr1_w, pr1_b.reshape(1, D), batf)

    r0_w, r0_b = params['readout'][0]
    r1_w, r1_b = params['readout'][1]
    out = _readout_call(g, r0_w, r0_b.reshape(1, D),
                        r1_w, r1_b.reshape(1, 1))
    return out.reshape(G)


# speedup vs baseline: 3.6098x; 1.0238x over previous
import functools

import jax
import jax.numpy as jnp
from jax import lax
from jax.experimental import pallas as pl
from jax.experimental.pallas import tpu as pltpu
from jax.experimental.pallas import tpu_sc as plsc

N = 10000
E = 320000
D = 128
DE = 16
G = 64

NC = 2
NS = 16
NW = NC * NS
EPW = E // NW
C = 80
NCH = EPW // C
NPT = N // NS
ZROWS = 125
ZROWS_S = 25
FR = 1000
NBLG = 4
NBLS = 3
EPS = E // NS
NCHS = EPS // C
DH = D // 2

BN = 2000
BE = 2560

_F32 = jnp.float32


def _full_spec(shape):
    return pl.BlockSpec(shape, lambda i: (0,) * len(shape))


def _dot(a, b):
    return jnp.dot(a, b, preferred_element_type=_F32)


def _silu(x):
    return x * jax.nn.sigmoid(x)


def _embed_body(x, w1, b1, w2, b2, wa, wb, h, p1, p2):
    t = _silu(_dot(x[...], w1[...]) + b1[...])
    hh = _dot(t, w2[...]) + b2[...]
    h[...] = hh
    p1[...] = _dot(hh, wa[...])
    p2[...] = _dot(hh, wb[...])


def _embed_call(x, w1, b1, w2, b2, wa, wb):
    return pl.pallas_call(
        _embed_body,
        grid=(N // BN,),
        in_specs=[pl.BlockSpec((BN, D), lambda i: (i, 0)),
                  _full_spec((D, D)), _full_spec((1, D)),
                  _full_spec((D, D)), _full_spec((1, D)),
                  _full_spec((D, D)), _full_spec((D, D))],
        out_specs=[pl.BlockSpec((BN, D), lambda i: (i, 0))] * 3,
        out_shape=[jax.ShapeDtypeStruct((N, D), _F32)] * 3,
    )(x, w1, b1, w2, b2, wa, wb)


def _edge_body(s, xe, w1c, b1, w2, b2, m):
    t = _silu(s[...] + _dot(xe[...], w1c[...]) + b1[...])
    m[...] = _silu(_dot(t, w2[...]) + b2[...])


def _edge_call(s, xe, w1c, b1, w2, b2):
    return pl.pallas_call(
        _edge_body,
        grid=(E // BE,),
        in_specs=[pl.BlockSpec((BE, D), lambda i: (i, 0)),
                  pl.BlockSpec((BE, DE), lambda i: (i, 0)),
                  _full_spec((DE, D)), _full_spec((1, D)),
                  _full_spec((D, D)), _full_spec((1, D))],
        out_specs=pl.BlockSpec((BE, D), lambda i: (i, 0)),
        out_shape=jax.ShapeDtypeStruct((E, D), _F32),
    )(s, xe, w1c, b1, w2, b2)


def _node_proj_body(h, pa, pb, wn1a, wn1b, bn1, wn2, bn2, wa, wb,
                    ho, p1, p2):
    u = _silu(_dot(h[...], wn1a[...]) + _dot(pa[...] + pb[...], wn1b[...])
              + bn1[...])
    hh = _dot(u, wn2[...]) + bn2[...]
    ho[...] = hh
    p1[...] = _dot(hh, wa[...])
    p2[...] = _dot(hh, wb[...])


def _node_proj_call(h, pa, pb, wn1a, wn1b, bn1, wn2, bn2, wa, wb):
    return pl.pallas_call(
        _node_proj_body,
        grid=(N // BN,),
        in_specs=[pl.BlockSpec((BN, D), lambda i: (i, 0))] * 3 + [
            _full_spec((D, D)), _full_spec((D, D)), _full_spec((1, D)),
            _full_spec((D, D)), _full_spec((1, D)),
            _full_spec((D, D)), _full_spec((D, D))],
        out_specs=[pl.BlockSpec((BN, D), lambda i: (i, 0))] * 3,
        out_shape=[jax.ShapeDtypeStruct((N, D), _F32)] * 3,
    )(h, pa, pb, wn1a, wn1b, bn1, wn2, bn2, wa, wb)


def _node_last_body(h, pa, pb, wn1a, wn1b, bn1, wn2, bn2, ho):
    u = _silu(_dot(h[...], wn1a[...]) + _dot(pa[...] + pb[...], wn1b[...])
              + bn1[...])
    ho[...] = _dot(u, wn2[...]) + bn2[...]


def _node_last_call(h, pa, pb, wn1a, wn1b, bn1, wn2, bn2):
    return pl.pallas_call(
        _node_last_body,
        grid=(N // BN,),
        in_specs=[pl.BlockSpec((BN, D), lambda i: (i, 0))] * 3 + [
            _full_spec((D, D)), _full_spec((D, D)), _full_spec((1, D)),
            _full_spec((D, D)), _full_spec((1, D))],
        out_specs=pl.BlockSpec((BN, D), lambda i: (i, 0)),
        out_shape=jax.ShapeDtypeStruct((N, D), _F32),
    )(h, pa, pb, wn1a, wn1b, bn1, wn2, bn2)


def _prepool_body(h, w1, b1, w2, b2, bat, o):
    t = _silu(_dot(h[...], w1[...]) + b1[...])
    hp = _dot(t, w2[...]) + b2[...]
    ids = bat[0]
    iota = lax.broadcasted_iota(jnp.int32, (G, BN), 0).astype(_F32)
    onehot = (iota == ids).astype(_F32)
    part = _dot(onehot, hp)

    @pl.when(pl.program_id(0) == 0)
    def _init():
        o[...] = jnp.zeros_like(o)

    o[...] += part


def _prepool_call(h, w1, b1, w2, b2, batf):
    return pl.pallas_call(
        _prepool_body,
        grid=(N // BN,),
        in_specs=[pl.BlockSpec((BN, D), lambda i: (i, 0)),
                  _full_spec((D, D)), _full_spec((1, D)),
                  _full_spec((D, D)), _full_spec((1, D)),
                  pl.BlockSpec((1, 1, BN), lambda i: (i, 0, 0))],
        out_specs=pl.BlockSpec((G, D), lambda i: (0, 0)),
        out_shape=jax.ShapeDtypeStruct((G, D), _F32),
    )(h, w1, b1, w2, b2, batf)


def _readout_body(g, w1, b1, w2, b2, o):
    u = _silu(_dot(g[...], w1[...]) + b1[...])
    o[...] = _dot(u, w2[...]) + b2[...]


def _readout_call(g, w1, b1, w2, b2):
    return pl.pallas_call(
        _readout_body,
        grid=(1,),
        in_specs=[_full_spec((G, D)),
                  _full_spec((D, D)), _full_spec((1, D)),
                  _full_spec((D, 1)), _full_spec((1, 1))],
        out_specs=_full_spec((G, 1)),
        out_shape=jax.ShapeDtypeStruct((G, 1), _F32),
    )(g, w1, b1, w2, b2)


@functools.cache
def _mesh():
    return plsc.VectorSubcoreMesh(core_axis_name="c", subcore_axis_name="s",
                                  num_cores=NC, num_subcores=NS)


def _sc_gather(p1, p2, snd, rcv):
    sds = jax.ShapeDtypeStruct((E, D), _F32)
    scr = []
    for _ in range(NBLG):
        scr += [pltpu.VMEM((C,), jnp.int32), pltpu.VMEM((C,), jnp.int32),
                pltpu.VMEM((C, D), _F32), pltpu.VMEM((C, D), _F32),
                pltpu.SemaphoreType.DMA, pltpu.SemaphoreType.DMA,
                pltpu.SemaphoreType.DMA]

    @functools.partial(pl.kernel, out_type=sds, mesh=_mesh(),
                       scratch_types=scr)
    def k(p1_h, p2_h, snd_h, rcv_h, s_h, *scratch):
        lanes = [scratch[i * 7:(i + 1) * 7] for i in range(NBLG)]
        wid = lax.axis_index("s") * NC + lax.axis_index("c")
        base = wid * EPW

        def chunk_group(first, nbl):
            ips = []
            for b in range(nbl):
                idx1, idx2, r1, r2, isem, gsem, wsem = lanes[b]
                sl = pl.ds(base + (first + b) * C, C)
                ips.append(
                    (pltpu.async_copy(snd_h.at[sl], idx1, isem),
                     pltpu.async_copy(rcv_h.at[sl], idx2, isem)))
            gps = []
            for b in range(nbl):
                idx1, idx2, r1, r2, isem, gsem, wsem = lanes[b]
                i1, i2 = ips[b]
                i1.wait()
                i2.wait()
                gps.append(
                    (pltpu.async_copy(p1_h.at[idx1], r1, gsem),
                     pltpu.async_copy(p2_h.at[idx2], r2, gsem)))
            wps = []
            for b in range(nbl):
                idx1, idx2, r1, r2, isem, gsem, wsem = lanes[b]
                g1, g2 = gps[b]
                g1.wait()
                g2.wait()

                def add_row(r, carry):
                    for c2 in range(D // 16):
                        cs = pl.ds(c2 * 16, 16)
                        r1[r, cs] = r1[r, cs] + r2[r, cs]
                    return carry

                lax.fori_loop(0, C, add_row, 0)
                sl = pl.ds(base + (first + b) * C, C)
                wps.append(pltpu.async_copy(r1, s_h.at[sl], wsem))
            for w1 in wps:
                w1.wait()

        def body(g, carry):
            chunk_group(g * NBLG, NBLG)
            return carry

        lax.fori_loop(0, NCH // NBLG, body, 0)
        if NCH % NBLG:
            chunk_group((NCH // NBLG) * NBLG, NCH % NBLG)

    return k(p1, p2, snd, rcv)


def _sc_scatter(m, rcv):
    scr = []
    for _ in range(NBLS):
        scr += [pltpu.VMEM((C,), jnp.int32), pltpu.VMEM((C, D), _F32),
                pltpu.SemaphoreType.DMA, pltpu.SemaphoreType.DMA,
                pltpu.SemaphoreType.DMA]
    scr += [pltpu.VMEM((ZROWS_S, D), _F32),
            pltpu.VMEM_SHARED((N, D), _F32)]

    @functools.partial(pl.kernel,
                       out_type=jax.ShapeDtypeStruct((NC * N, D), _F32),
                       mesh=_mesh(), scratch_types=scr)
    def k(m_h, rcv_h, out_h, *scratch):
        lanes = [scratch[i * 5:(i + 1) * 5] for i in range(NBLS)]
        zbuf, aggr = scratch[5 * NBLS], scratch[5 * NBLS + 1]
        cid = lax.axis_index("c")
        sid = lax.axis_index("s")
        base = (sid * NC + cid) * EPW

        def zrow(r, carry):
            def zcol(c2, carry2):
                zbuf[r, pl.ds(c2 * 16, 16)] = jnp.zeros((16,), _F32)
                return carry2
            return lax.fori_loop(0, D // 16, zcol, carry)

        lax.fori_loop(0, ZROWS_S, zrow, 0)

        def zcopy(j, carry):
            pltpu.sync_copy(zbuf,
                            aggr.at[pl.ds(sid * NPT + j * ZROWS_S, ZROWS_S)])
            return carry

        lax.fori_loop(0, NPT // ZROWS_S, zcopy, 0)
        plsc.subcore_barrier()

        def chunk_group(first, nbl):
            ips = []
            for b in range(nbl):
                idx, rows, isem, rsem, ssem = lanes[b]
                sl = pl.ds(base + (first + b) * C, C)
                ips.append(
                    (pltpu.async_copy(rcv_h.at[sl], idx, isem),
                     pltpu.async_copy(m_h.at[sl], rows, rsem)))
            sps = []
            for b in range(nbl):
                idx, rows, isem, rsem, ssem = lanes[b]
                i1, r1 = ips[b]
                i1.wait()
                r1.wait()
                sps.append(pltpu.async_copy(rows, aggr.at[idx], ssem,
                                            add=True))
            for sp in sps:
                sp.wait()

        def body(g, carry):
            chunk_group(g * NBLS, NBLS)
            return carry

        lax.fori_loop(0, NCH // NBLS, body, 0)
        if NCH % NBLS:
            chunk_group((NCH // NBLS) * NBLS, NCH % NBLS)
        plsc.subcore_barrier()

        @pl.when(sid < N // FR)
        def _flush():
            pltpu.sync_copy(aggr.at[pl.ds(sid * FR, FR)],
                            out_h.at[pl.ds(cid * N + sid * FR, FR)])

    return k(m, rcv)


def kernel(x_nodes, x_edges, edge_index, batch, params):
    sender = edge_index[0]
    receiver = edge_index[1]
    layers = params['layers']

    e0_w, e0_b = params['embed'][0]
    e1_w, e1_b = params['embed'][1]
    w1_first = layers[0]['edge'][0][0]
    h, p1, p2 = _embed_call(
        x_nodes, e0_w, e0_b.reshape(1, D), e1_w, e1_b.reshape(1, D),
        w1_first[:D], w1_first[D:2 * D])

    for li, lp in enumerate(layers):
        we1, be1 = lp['edge'][0]
        we2, be2 = lp['edge'][1]
        s = _sc_gather(p1, p2, sender, receiver)
        m = _edge_call(s, x_edges, we1[2 * D:], be1.reshape(1, D),
                       we2, be2.reshape(1, D))
        parts = _sc_scatter(m, receiver).reshape(2, N, D)
        wn1, bn1 = lp['node'][0]
        wn2, bn2 = lp['node'][1]
        if li + 1 < len(layers):
            w1_next = layers[li + 1]['edge'][0][0]
            h, p1, p2 = _node_proj_call(
                h, parts[0], parts[1], wn1[:D], wn1[D:], bn1.reshape(1, D),
                wn2, bn2.reshape(1, D), w1_next[:D], w1_next[D:2 * D])
        else:
            h = _node_last_call(
                h, parts[0], parts[1], wn1[:D], wn1[D:], bn1.reshape(1, D),
                wn2, bn2.reshape(1, D))

    pr0_w, pr0_b = params['pre'][0]
    pr1_w, pr1_b = params['pre'][1]
    batf = batch.astype(_F32).reshape(N // BN, 1, BN)
    g = _prepool_call(h, pr0_w, pr0_b.reshape(1, D),
                      pr1_w, pr1_b.reshape(1, D), batf)

    r0_w, r0_b = params['readout'][0]
    r1_w, r1_b = params['readout'][1]
    out = _readout_call(g, r0_w, r0_b.reshape(1, D),
                        r1_w, r1_b.reshape(1, 1))
    return out.reshape(G)


# speedup vs baseline: 4.1292x; 1.1439x over previous
import functools

import jax
import jax.numpy as jnp
from jax import lax
from jax.experimental import pallas as pl
from jax.experimental.pallas import tpu as pltpu
from jax.experimental.pallas import tpu_sc as plsc

N = 10000
E = 320000
D = 128
DE = 16
G = 64

NC = 2
NS = 16
NW = NC * NS
EPW = E // NW
C = 80
NCH = EPW // C
NPT = N // NS
ZROWS = 125
ZROWS_S = 25
FR = 1000
NBLG = 4
NBLS = 3
NCHA = 62
NCHB = 63
EPS = E // NS
NCHS = EPS // C
DH = D // 2

BN = 2000
BE = 2560

_F32 = jnp.float32


def _full_spec(shape):
    return pl.BlockSpec(shape, lambda i: (0,) * len(shape))


def _dot(a, b):
    return jnp.dot(a, b, preferred_element_type=_F32)


def _silu(x):
    return x * jax.nn.sigmoid(x)


def _embed_body(x, w1, b1, w2, b2, wa, wb, h, p1, p2):
    t = _silu(_dot(x[...], w1[...]) + b1[...])
    hh = _dot(t, w2[...]) + b2[...]
    h[...] = hh
    p1[...] = _dot(hh, wa[...])
    p2[...] = _dot(hh, wb[...])


def _embed_call(x, w1, b1, w2, b2, wa, wb):
    return pl.pallas_call(
        _embed_body,
        grid=(N // BN,),
        in_specs=[pl.BlockSpec((BN, D), lambda i: (i, 0)),
                  _full_spec((D, D)), _full_spec((1, D)),
                  _full_spec((D, D)), _full_spec((1, D)),
                  _full_spec((D, D)), _full_spec((D, D))],
        out_specs=[pl.BlockSpec((BN, D), lambda i: (i, 0))] * 3,
        out_shape=[jax.ShapeDtypeStruct((N, D), _F32)] * 3,
    )(x, w1, b1, w2, b2, wa, wb)


def _edge_body(s, xe, w1c, b1, w2, b2, m):
    t = _silu(s[...] + _dot(xe[...], w1c[...]) + b1[...])
    m[...] = _silu(_dot(t, w2[...]) + b2[...])


def _edge_call(s, xe, w1c, b1, w2, b2):
    ne = s.shape[0]
    return pl.pallas_call(
        _edge_body,
        grid=(ne // BE,),
        in_specs=[pl.BlockSpec((BE, D), lambda i: (i, 0)),
                  pl.BlockSpec((BE, DE), lambda i: (i, 0)),
                  _full_spec((DE, D)), _full_spec((1, D)),
                  _full_spec((D, D)), _full_spec((1, D))],
        out_specs=pl.BlockSpec((BE, D), lambda i: (i, 0)),
        out_shape=jax.ShapeDtypeStruct((ne, D), _F32),
    )(s, xe, w1c, b1, w2, b2)


def _node_proj_body(h, pa, pb, pc, pd, wn1a, wn1b, bn1, wn2, bn2, wa, wb,
                    ho, p1, p2):
    aggr = (pa[...] + pb[...]) + (pc[...] + pd[...])
    u = _silu(_dot(h[...], wn1a[...]) + _dot(aggr, wn1b[...]) + bn1[...])
    hh = _dot(u, wn2[...]) + bn2[...]
    ho[...] = hh
    p1[...] = _dot(hh, wa[...])
    p2[...] = _dot(hh, wb[...])


def _node_proj_call(h, pa, pb, pc, pd, wn1a, wn1b, bn1, wn2, bn2, wa, wb):
    return pl.pallas_call(
        _node_proj_body,
        grid=(N // BN,),
        in_specs=[pl.BlockSpec((BN, D), lambda i: (i, 0))] * 5 + [
            _full_spec((D, D)), _full_spec((D, D)), _full_spec((1, D)),
            _full_spec((D, D)), _full_spec((1, D)),
            _full_spec((D, D)), _full_spec((D, D))],
        out_specs=[pl.BlockSpec((BN, D), lambda i: (i, 0))] * 3,
        out_shape=[jax.ShapeDtypeStruct((N, D), _F32)] * 3,
    )(h, pa, pb, pc, pd, wn1a, wn1b, bn1, wn2, bn2, wa, wb)


def _node_last_body(h, pa, pb, pc, pd, wn1a, wn1b, bn1, wn2, bn2, ho):
    aggr = (pa[...] + pb[...]) + (pc[...] + pd[...])
    u = _silu(_dot(h[...], wn1a[...]) + _dot(aggr, wn1b[...]) + bn1[...])
    ho[...] = _dot(u, wn2[...]) + bn2[...]


def _node_last_call(h, pa, pb, pc, pd, wn1a, wn1b, bn1, wn2, bn2):
    return pl.pallas_call(
        _node_last_body,
        grid=(N // BN,),
        in_specs=[pl.BlockSpec((BN, D), lambda i: (i, 0))] * 5 + [
            _full_spec((D, D)), _full_spec((D, D)), _full_spec((1, D)),
            _full_spec((D, D)), _full_spec((1, D))],
        out_specs=pl.BlockSpec((BN, D), lambda i: (i, 0)),
        out_shape=jax.ShapeDtypeStruct((N, D), _F32),
    )(h, pa, pb, pc, pd, wn1a, wn1b, bn1, wn2, bn2)


def _prepool_body(h, w1, b1, w2, b2, bat, o):
    t = _silu(_dot(h[...], w1[...]) + b1[...])
    hp = _dot(t, w2[...]) + b2[...]
    ids = bat[0]
    iota = lax.broadcasted_iota(jnp.int32, (G, BN), 0).astype(_F32)
    onehot = (iota == ids).astype(_F32)
    part = _dot(onehot, hp)

    @pl.when(pl.program_id(0) == 0)
    def _init():
        o[...] = jnp.zeros_like(o)

    o[...] += part


def _prepool_call(h, w1, b1, w2, b2, batf):
    return pl.pallas_call(
        _prepool_body,
        grid=(N // BN,),
        in_specs=[pl.BlockSpec((BN, D), lambda i: (i, 0)),
                  _full_spec((D, D)), _full_spec((1, D)),
                  _full_spec((D, D)), _full_spec((1, D)),
                  pl.BlockSpec((1, 1, BN), lambda i: (i, 0, 0))],
        out_specs=pl.BlockSpec((G, D), lambda i: (0, 0)),
        out_shape=jax.ShapeDtypeStruct((G, D), _F32),
    )(h, w1, b1, w2, b2, batf)


def _readout_body(g, w1, b1, w2, b2, o):
    u = _silu(_dot(g[...], w1[...]) + b1[...])
    o[...] = _dot(u, w2[...]) + b2[...]


def _readout_call(g, w1, b1, w2, b2):
    return pl.pallas_call(
        _readout_body,
        grid=(1,),
        in_specs=[_full_spec((G, D)),
                  _full_spec((D, D)), _full_spec((1, D)),
                  _full_spec((D, 1)), _full_spec((1, 1))],
        out_specs=_full_spec((G, 1)),
        out_shape=jax.ShapeDtypeStruct((G, 1), _F32),
    )(g, w1, b1, w2, b2)


@functools.cache
def _mesh():
    return plsc.VectorSubcoreMesh(core_axis_name="c", subcore_axis_name="s",
                                  num_cores=NC, num_subcores=NS)


def _sc_gather(p1, p2, snd, rcv, nch):
    epw = nch * C
    sds = jax.ShapeDtypeStruct((NW * epw, D), _F32)
    scr = []
    for _ in range(NBLG):
        scr += [pltpu.VMEM((C,), jnp.int32), pltpu.VMEM((C,), jnp.int32),
                pltpu.VMEM((C, D), _F32), pltpu.VMEM((C, D), _F32),
                pltpu.SemaphoreType.DMA, pltpu.SemaphoreType.DMA,
                pltpu.SemaphoreType.DMA]

    @functools.partial(pl.kernel, out_type=sds, mesh=_mesh(),
                       scratch_types=scr)
    def k(p1_h, p2_h, snd_h, rcv_h, s_h, *scratch):
        lanes = [scratch[i * 7:(i + 1) * 7] for i in range(NBLG)]
        wid = lax.axis_index("s") * NC + lax.axis_index("c")
        base = wid * epw

        def chunk_group(first, nbl):
            ips = []
            for b in range(nbl):
                idx1, idx2, r1, r2, isem, gsem, wsem = lanes[b]
                sl = pl.ds(base + (first + b) * C, C)
                ips.append(
                    (pltpu.async_copy(snd_h.at[sl], idx1, isem),
                     pltpu.async_copy(rcv_h.at[sl], idx2, isem)))
            gps = []
            for b in range(nbl):
                idx1, idx2, r1, r2, isem, gsem, wsem = lanes[b]
                i1, i2 = ips[b]
                i1.wait()
                i2.wait()
                gps.append(
                    (pltpu.async_copy(p1_h.at[idx1], r1, gsem),
                     pltpu.async_copy(p2_h.at[idx2], r2, gsem)))
            wps = []
            for b in range(nbl):
                idx1, idx2, r1, r2, isem, gsem, wsem = lanes[b]
                g1, g2 = gps[b]
                g1.wait()
                g2.wait()

                def add_row(r, carry):
                    for c2 in range(D // 16):
                        cs = pl.ds(c2 * 16, 16)
                        r1[r, cs] = r1[r, cs] + r2[r, cs]
                    return carry

                lax.fori_loop(0, C, add_row, 0)
                sl = pl.ds(base + (first + b) * C, C)
                wps.append(pltpu.async_copy(r1, s_h.at[sl], wsem))
            for w1 in wps:
                w1.wait()

        def body(g, carry):
            chunk_group(g * NBLG, NBLG)
            return carry

        lax.fori_loop(0, nch // NBLG, body, 0)
        if nch % NBLG:
            chunk_group((nch // NBLG) * NBLG, nch % NBLG)

    return k(p1, p2, snd, rcv)


def _sc_scatter(m, rcv, nch):
    scr = []
    for _ in range(NBLS):
        scr += [pltpu.VMEM((C,), jnp.int32), pltpu.VMEM((C, D), _F32),
                pltpu.SemaphoreType.DMA, pltpu.SemaphoreType.DMA,
                pltpu.SemaphoreType.DMA]
    scr += [pltpu.VMEM((ZROWS_S, D), _F32),
            pltpu.VMEM_SHARED((N, D), _F32)]

    @functools.partial(pl.kernel,
                       out_type=jax.ShapeDtypeStruct((NC * N, D), _F32),
                       mesh=_mesh(), scratch_types=scr)
    def k(m_h, rcv_h, out_h, *scratch):
        lanes = [scratch[i * 5:(i + 1) * 5] for i in range(NBLS)]
        zbuf, aggr = scratch[5 * NBLS], scratch[5 * NBLS + 1]
        cid = lax.axis_index("c")
        sid = lax.axis_index("s")
        base = (sid * NC + cid) * nch * C

        def zrow(r, carry):
            def zcol(c2, carry2):
                zbuf[r, pl.ds(c2 * 16, 16)] = jnp.zeros((16,), _F32)
                return carry2
            return lax.fori_loop(0, D // 16, zcol, carry)

        lax.fori_loop(0, ZROWS_S, zrow, 0)

        def zcopy(j, carry):
            pltpu.sync_copy(zbuf,
                            aggr.at[pl.ds(sid * NPT + j * ZROWS_S, ZROWS_S)])
            return carry

        lax.fori_loop(0, NPT // ZROWS_S, zcopy, 0)
        plsc.subcore_barrier()

        def chunk_group(first, nbl):
            ips = []
            for b in range(nbl):
                idx, rows, isem, rsem, ssem = lanes[b]
                sl = pl.ds(base + (first + b) * C, C)
                ips.append(
                    (pltpu.async_copy(rcv_h.at[sl], idx, isem),
                     pltpu.async_copy(m_h.at[sl], rows, rsem)))
            sps = []
            for b in range(nbl):
                idx, rows, isem, rsem, ssem = lanes[b]
                i1, r1 = ips[b]
                i1.wait()
                r1.wait()
                sps.append(pltpu.async_copy(rows, aggr.at[idx], ssem,
                                            add=True))
            for sp in sps:
                sp.wait()

        def body(g, carry):
            chunk_group(g * NBLS, NBLS)
            return carry

        lax.fori_loop(0, nch // NBLS, body, 0)
        if nch % NBLS:
            chunk_group((nch // NBLS) * NBLS, nch % NBLS)
        plsc.subcore_barrier()

        @pl.when(sid < N // FR)
        def _flush():
            pltpu.sync_copy(aggr.at[pl.ds(sid * FR, FR)],
                            out_h.at[pl.ds(cid * N + sid * FR, FR)])

    return k(m, rcv)


def kernel(x_nodes, x_edges, edge_index, batch, params):
    na = (NCHA * C) * NW
    snd_a, snd_b = edge_index[0, :na], edge_index[0, na:]
    rcv_a, rcv_b = edge_index[1, :na], edge_index[1, na:]
    xe_a, xe_b = x_edges[:na], x_edges[na:]
    layers = params['layers']

    e0_w, e0_b = params['embed'][0]
    e1_w, e1_b = params['embed'][1]
    w1_first = layers[0]['edge'][0][0]
    h, p1, p2 = _embed_call(
        x_nodes, e0_w, e0_b.reshape(1, D), e1_w, e1_b.reshape(1, D),
        w1_first[:D], w1_first[D:2 * D])

    for li, lp in enumerate(layers):
        we1, be1 = lp['edge'][0]
        we2, be2 = lp['edge'][1]
        s_a = _sc_gather(p1, p2, snd_a, rcv_a, NCHA)
        s_b = _sc_gather(p1, p2, snd_b, rcv_b, NCHB)
        m_a = _edge_call(s_a, xe_a, we1[2 * D:], be1.reshape(1, D),
                         we2, be2.reshape(1, D))
        m_b = _edge_call(s_b, xe_b, we1[2 * D:], be1.reshape(1, D),
                         we2, be2.reshape(1, D))
        parts_a = _sc_scatter(m_a, rcv_a, NCHA).reshape(2, N, D)
        parts_b = _sc_scatter(m_b, rcv_b, NCHB).reshape(2, N, D)
        wn1, bn1 = lp['node'][0]
        wn2, bn2 = lp['node'][1]
        if li + 1 < len(layers):
            w1_next = layers[li + 1]['edge'][0][0]
            h, p1, p2 = _node_proj_call(
                h, parts_a[0], parts_a[1], parts_b[0], parts_b[1],
                wn1[:D], wn1[D:], bn1.reshape(1, D),
                wn2, bn2.reshape(1, D), w1_next[:D], w1_next[D:2 * D])
        else:
            h = _node_last_call(
                h, parts_a[0], parts_a[1], parts_b[0], parts_b[1],
                wn1[:D], wn1[D:], bn1.reshape(1, D),
                wn2, bn2.reshape(1, D))

    pr0_w, pr0_b = params['pre'][0]
    pr1_w, pr1_b = params['pre'][1]
    batf = batch.astype(_F32).reshape(N // BN, 1, BN)
    g = _prepool_call(h, pr0_w, pr0_b.reshape(1, D),
                      pr1_w, pr1_b.reshape(1, D), batf)

    r0_w, r0_b = params['readout'][0]
    r1_w, r1_b = params['readout'][1]
    out = _readout_call(g, r0_w, r0_b.reshape(1, D),
                        r1_w, r1_b.reshape(1, 1))
    return out.reshape(G)
